# baseline (speedup 1.0000x reference)
import dataclasses

import jax
import jax.numpy as jnp
from jax import lax
from jax.experimental import pallas as pl
from jax.experimental.pallas import tpu as pltpu
from jax.experimental.pallas import tpu_sc as plsc

N = 10000
E = 320000
ND = 128
H = 64
W = 128

_NC = 2
_NS = 16
_NW = _NC * _NS
_EB = 80
_EPT = E // _NW
_NB = _EPT // _EB
_NRB = N // _EB
_NRB_PT = -(-_NRB // _NS)
_L = 16

_D = 4
_DH = 6

_mesh = plsc.VectorSubcoreMesh(core_axis_name="c", subcore_axis_name="s",
                               num_cores=_NC, num_subcores=_NS)

_cp = pltpu.CompilerParams()
if "needs_layout_passes" in pltpu.CompilerParams.__dataclass_fields__:
  _cp = dataclasses.replace(_cp, needs_layout_passes=False)


def _sage_body(h_hbm, src_hbm, dst_hbm, agg_out, src_r, dst_r, rows, agg_sh,
               *sems):
  sem_is = sems[:_D]
  sem_id = sems[_D:2 * _D]
  sem_g = sems[2 * _D:3 * _D]
  sem_s = sems[3 * _D:4 * _D]
  ci = lax.axis_index("c")
  si = lax.axis_index("s")
  wid = ci * _NS + si

  @pl.loop(0, _EB)
  def _zero_rows(r):
    for j in range(W // _L):
      rows[0, r, pl.ds(j * _L, _L)] = jnp.zeros((_L,), jnp.float32)

  @pl.loop(0, _NRB_PT)
  def _zero_sh(k):
    blk = si + k * _NS

    @pl.when(blk < _NRB)
    def _():
      rr = pl.multiple_of(blk * _EB, _EB)
      pltpu.sync_copy(rows.at[0], agg_sh.at[pl.ds(rr, _EB)])

  def _off(c):
    return pl.multiple_of(wid * _EPT + c * _EB, _EB)

  def fire_is(c, r):
    pltpu.async_copy(src_hbm.at[pl.ds(_off(c), _EB)], src_r.at[r], sem_is[r])

  def wait_is(c, r):
    pltpu.make_async_copy(src_hbm.at[pl.ds(_off(c), _EB)], src_r.at[r],
                          sem_is[r]).wait()

  def fire_id(c, r):
    pltpu.async_copy(dst_hbm.at[pl.ds(_off(c), _EB)], dst_r.at[r], sem_id[r])

  def wait_id(c, r):
    pltpu.make_async_copy(dst_hbm.at[pl.ds(_off(c), _EB)], dst_r.at[r],
                          sem_id[r]).wait()

  def fire_g(c, r):
    pltpu.async_copy(h_hbm.at[src_r.at[r]], rows.at[r], sem_g[r])

  def wait_g(c, r):
    pltpu.make_async_copy(h_hbm.at[src_r.at[r]], rows.at[r], sem_g[r]).wait()

  def fire_s(c, r):
    pltpu.async_copy(rows.at[r], agg_sh.at[dst_r.at[r]], sem_s[r], add=True)

  def wait_s(c, r):
    pltpu.make_async_copy(rows.at[r], agg_sh.at[dst_r.at[r]],
                          sem_s[r]).wait()

  for r in range(_D):
    fire_is(r, r)
  for r in range(2):
    fire_id(r, r)
    wait_is(r, r)
    fire_g(r, r)

  plsc.subcore_barrier()

  @pl.loop(0, -(-_NB // _D))
  def _main(i):
    for r in range(_D):
      c = i * _D + r

      @pl.when(c < _NB)
      def _():
        wait_g(c, r)
        wait_id(c, r)
        fire_s(c, r)

        @pl.when(c + 2 < _NB)
        def _():
          r2 = (r + 2) % _D

          @pl.when(c >= 2)
          def _():
            wait_s(c - 2, r2)

          fire_id(c + 2, r2)
          wait_is(c + 2, r2)
          fire_g(c + 2, r2)

        @pl.when(c + _D < _NB)
        def _():
          fire_is(c + _D, r)

  for r in range(_D):
    wait_s(0, r)

  plsc.subcore_barrier()

  @pl.loop(0, _NRB_PT)
  def _copy_out(k):
    blk = si + k * _NS

    @pl.when(blk < _NRB)
    def _():
      rr = pl.multiple_of(blk * _EB, _EB)
      pltpu.sync_copy(agg_sh.at[pl.ds(rr, _EB)],
                      agg_out.at[ci, pl.ds(rr, _EB)])


_sage_sc = pl.kernel(
    _sage_body,
    out_type=jax.ShapeDtypeStruct((_NC, N, W), jnp.float32),
    mesh=_mesh,
    scratch_types=[
        pltpu.VMEM((_D, _EB), jnp.int32),
        pltpu.VMEM((_D, _EB), jnp.int32),
        pltpu.VMEM((_D, _EB, W), jnp.float32),
        pltpu.VMEM_SHARED((N, W), jnp.float32),
    ] + [pltpu.SemaphoreType.DMA] * (4 * _D))


def _head_body(a_hbm, b_hbm, src_hbm, dst_hbm, w2_hbm, b2_hbm, out_hbm,
               ps_hbm, src_r, dst_r, ra, rb, tmp, lv, wv, bv, psum, *sems):
  sem_is = sems[:_DH]
  sem_id = sems[_DH:2 * _DH]
  sem_a = sems[2 * _DH:3 * _DH]
  sem_b = sems[3 * _DH:4 * _DH]
  sem_w = sems[4 * _DH:5 * _DH]
  ci = lax.axis_index("c")
  si = lax.axis_index("s")
  wid = ci * _NS + si
  pltpu.sync_copy(w2_hbm, wv)
  pltpu.sync_copy(b2_hbm, bv)
  w = [wv[pl.ds(j * _L, _L)] for j in range(H // _L)]
  b2 = bv[pl.ds(0, _L)]
  psum[0, pl.ds(0, _L)] = jnp.zeros((_L,), jnp.float32)
  iota16 = lax.iota(jnp.int32, _L)
  base = wid * _EPT

  def _off(c):
    return pl.multiple_of(base + c * _EB, _EB)

  def fire_i(c, r):
    pltpu.async_copy(src_hbm.at[pl.ds(_off(c), _EB)], src_r.at[r], sem_is[r])
    pltpu.async_copy(dst_hbm.at[pl.ds(_off(c), _EB)], dst_r.at[r], sem_id[r])

  def wait_i(c, r):
    pltpu.make_async_copy(src_hbm.at[pl.ds(_off(c), _EB)], src_r.at[r],
                          sem_is[r]).wait()
    pltpu.make_async_copy(dst_hbm.at[pl.ds(_off(c), _EB)], dst_r.at[r],
                          sem_id[r]).wait()

  def fire_g(c, r):
    pltpu.async_copy(a_hbm.at[src_r.at[r]], ra.at[r], sem_a[r])
    pltpu.async_copy(b_hbm.at[dst_r.at[r]], rb.at[r], sem_b[r])

  def wait_g(c, r):
    pltpu.make_async_copy(a_hbm.at[src_r.at[r]], ra.at[r], sem_a[r]).wait()
    pltpu.make_async_copy(b_hbm.at[dst_r.at[r]], rb.at[r], sem_b[r]).wait()

  def wait_w(r):
    pltpu.make_async_copy(lv.at[r], out_hbm.at[pl.ds(0, _EB)],
                          sem_w[r]).wait()

  for r in range(_DH):
    fire_i(r, r)
  for r in range(_DH - 2):
    wait_i(r, r)
    fire_g(r, r)

  @pl.loop(0, -(-_NB // _DH))
  def _main(i):
    for r in range(_DH):
      c = i * _DH + r

      @pl.when(c < _NB)
      def _():
        wait_g(c, r)

        @pl.when(c + _DH < _NB)
        def _():
          fire_i(c + _DH, r)

        @pl.when(c + _DH - 2 < _NB)
        def _():
          r2 = (r + _DH - 2) % _DH
          wait_i(c + _DH - 2, r2)
          fire_g(c + _DH - 2, r2)

        @pl.when(c >= _DH)
        def _():
          wait_w(r)

        @pl.loop(0, _EB // _L)
        def _group(g):
          for i2 in range(_L):
            acc = jnp.zeros((_L,), jnp.float32)
            for j in range(H // _L):
              cj = jnp.maximum(
                  ra[r, g * _L + i2, pl.ds(j * _L, _L)]
                  + rb[r, g * _L + i2, pl.ds(j * _L, _L)], 0.0)
              acc = acc + cj * w[j]
            tmp[i2, pl.ds(0, _L)] = acc
          tot = jnp.zeros((_L,), jnp.float32)
          for j in range(_L):
            tot = tot + plsc.load_gather(
                tmp, [iota16, jnp.full((_L,), j, jnp.int32)])
          s = 1.0 / (1.0 + jnp.exp(-(tot + b2)))
          psum[0, pl.ds(0, _L)] = psum[0, pl.ds(0, _L)] + s
          lv[r, pl.ds(g * _L, _L)] = s

        pltpu.async_copy(lv.at[r], out_hbm.at[pl.ds(_off(c), _EB)], sem_w[r])

  for r in range(_DH):
    wait_w(r)
  pltpu.sync_copy(psum, ps_hbm.at[ci, si])


_head_sc = pl.kernel(
    _head_body,
    out_type=[
        jax.ShapeDtypeStruct((E,), jnp.float32),
        jax.ShapeDtypeStruct((_NC, _NS, 1, _L), jnp.float32),
    ],
    mesh=_mesh,
    compiler_params=_cp,
    scratch_types=[
        pltpu.VMEM((_DH, _EB), jnp.int32),
        pltpu.VMEM((_DH, _EB), jnp.int32),
        pltpu.VMEM((_DH, _EB, W), jnp.float32),
        pltpu.VMEM((_DH, _EB, W), jnp.float32),
        pltpu.VMEM((_L, _L), jnp.float32),
        pltpu.VMEM((_DH, _EB), jnp.float32),
        pltpu.VMEM((H,), jnp.float32),
        pltpu.VMEM((_L,), jnp.float32),
        pltpu.VMEM((1, _L), jnp.float32),
    ] + [pltpu.SemaphoreType.DMA] * (5 * _DH))


def _pad_cols(h):
  n = h.shape[0]
  ind = (lax.broadcasted_iota(jnp.int32, (n, W - H), 1) == 0).astype(
      jnp.float32)
  return jnp.concatenate([h, ind], axis=1)


def _enc_body(x_ref, wn_ref, bn_ref, o_ref):
  h = jnp.maximum(
      jnp.dot(x_ref[...], wn_ref[...], preferred_element_type=jnp.float32)
      + bn_ref[...], 0.0)
  o_ref[...] = _pad_cols(h)


def _mean_agg(p0_ref, p1_ref):
  p = p0_ref[...] + p1_ref[...]
  deg = jnp.maximum(p[:, H:H + 1], 1.0)
  return p[:, :H] / deg


def _combine_body(p0, p1, hp, wl, bl, wr, o):
  mean = _mean_agg(p0, p1)
  h = jnp.maximum(
      jnp.dot(mean, wl[...], preferred_element_type=jnp.float32) + bl[...]
      + jnp.dot(hp[...][:, :H], wr[...], preferred_element_type=jnp.float32),
      0.0)
  o[...] = _pad_cols(h)


def _combine_head_body(p0, p1, hp, wl, bl, wr, wha, whb, bh, o_h, o_a, o_b):
  mean = _mean_agg(p0, p1)
  h2 = jnp.maximum(
      jnp.dot(mean, wl[...], preferred_element_type=jnp.float32) + bl[...]
      + jnp.dot(hp[...][:, :H], wr[...], preferred_element_type=jnp.float32),
      0.0)
  o_h[...] = h2
  a = jnp.dot(h2, wha[...], preferred_element_type=jnp.float32) + bh[...]
  b = jnp.dot(h2, whb[...], preferred_element_type=jnp.float32)
  zpad = jnp.zeros((h2.shape[0], W - H), jnp.float32)
  o_a[...] = jnp.concatenate([a, zpad], axis=1)
  o_b[...] = jnp.concatenate([b, zpad], axis=1)


def _final_body(ps_ref, o_p):
  o_p[...] = (jnp.sum(ps_ref[...]) / E).reshape(1, 1)


def kernel(x, edge_index, edge_attr, Wn, bn, We, be, Wl0, bl0, Wr0, Wl1, bl1,
           Wr1, Wh1, bh1, Wh2, bh2):
  src = edge_index[0]
  dst = edge_index[1]
  f32 = jnp.float32

  h0p = pl.pallas_call(
      _enc_body, out_shape=jax.ShapeDtypeStruct((N, W), f32))(
          x, Wn, bn.reshape(1, H))

  agg0 = _sage_sc(h0p, src, dst)

  h1p = pl.pallas_call(
      _combine_body, out_shape=jax.ShapeDtypeStruct((N, W), f32))(
          agg0[0], agg0[1], h0p, Wl0, bl0.reshape(1, H), Wr0)

  agg1 = _sage_sc(h1p, src, dst)

  h2, ap, bp = pl.pallas_call(
      _combine_head_body,
      out_shape=[
          jax.ShapeDtypeStruct((N, H), f32),
          jax.ShapeDtypeStruct((N, W), f32),
          jax.ShapeDtypeStruct((N, W), f32),
      ])(agg1[0], agg1[1], h1p, Wl1, bl1.reshape(1, H), Wr1, Wh1[:H],
         Wh1[H:], bh1.reshape(1, H))

  scores, psums = _head_sc(ap, bp, src, dst, Wh2.reshape(H),
                           jnp.broadcast_to(bh2, (_L,)))

  psum = pl.pallas_call(
      _final_body, out_shape=jax.ShapeDtypeStruct((1, 1), f32))(
          psums.reshape(_NW, _L))

  return scores, psum[0, 0], h2


# speedup vs baseline: 1.0078x; 1.0078x over previous
import dataclasses

import jax
import jax.numpy as jnp
from jax import lax
from jax.experimental import pallas as pl
from jax.experimental.pallas import tpu as pltpu
from jax.experimental.pallas import tpu_sc as plsc

N = 10000
E = 320000
ND = 128
H = 64
W = 128

_NC = 2
_NS = 16
_NW = _NC * _NS
_EB = 80
_EPT = E // _NW
_NB = _EPT // _EB
_NRB = N // _EB
_NRB_PT = -(-_NRB // _NS)
_L = 16

_D = 4
_DH = 4

_mesh = plsc.VectorSubcoreMesh(core_axis_name="c", subcore_axis_name="s",
                               num_cores=_NC, num_subcores=_NS)

_cp = pltpu.CompilerParams()
if "needs_layout_passes" in pltpu.CompilerParams.__dataclass_fields__:
  _cp = dataclasses.replace(_cp, needs_layout_passes=False)


def _sage_body(h_hbm, src_hbm, dst_hbm, agg_out, src_r, dst_r, rows, agg_sh,
               *sems):
  sem_is = sems[:_D]
  sem_id = sems[_D:2 * _D]
  sem_g = sems[2 * _D:3 * _D]
  sem_s = sems[3 * _D:4 * _D]
  ci = lax.axis_index("c")
  si = lax.axis_index("s")
  wid = ci * _NS + si

  @pl.loop(0, _EB)
  def _zero_rows(r):
    for j in range(W // _L):
      rows[0, r, pl.ds(j * _L, _L)] = jnp.zeros((_L,), jnp.float32)

  @pl.loop(0, _NRB_PT)
  def _zero_sh(k):
    blk = si + k * _NS

    @pl.when(blk < _NRB)
    def _():
      rr = pl.multiple_of(blk * _EB, _EB)
      pltpu.sync_copy(rows.at[0], agg_sh.at[pl.ds(rr, _EB)])

  def _off(c):
    return pl.multiple_of(wid * _EPT + c * _EB, _EB)

  def fire_is(c, r):
    pltpu.async_copy(src_hbm.at[pl.ds(_off(c), _EB)], src_r.at[r], sem_is[r])

  def wait_is(c, r):
    pltpu.make_async_copy(src_hbm.at[pl.ds(_off(c), _EB)], src_r.at[r],
                          sem_is[r]).wait()

  def fire_id(c, r):
    pltpu.async_copy(dst_hbm.at[pl.ds(_off(c), _EB)], dst_r.at[r], sem_id[r])

  def wait_id(c, r):
    pltpu.make_async_copy(dst_hbm.at[pl.ds(_off(c), _EB)], dst_r.at[r],
                          sem_id[r]).wait()

  def fire_g(c, r):
    pltpu.async_copy(h_hbm.at[src_r.at[r]], rows.at[r], sem_g[r])

  def wait_g(c, r):
    pltpu.make_async_copy(h_hbm.at[src_r.at[r]], rows.at[r], sem_g[r]).wait()

  def fire_s(c, r):
    pltpu.async_copy(rows.at[r], agg_sh.at[dst_r.at[r]], sem_s[r], add=True)

  def wait_s(c, r):
    pltpu.make_async_copy(rows.at[r], agg_sh.at[dst_r.at[r]],
                          sem_s[r]).wait()

  for r in range(_D):
    fire_is(r, r)
  for r in range(2):
    fire_id(r, r)
    wait_is(r, r)
    fire_g(r, r)

  plsc.subcore_barrier()

  @pl.loop(0, -(-_NB // _D))
  def _main(i):
    for r in range(_D):
      c = i * _D + r

      @pl.when(c < _NB)
      def _():
        wait_g(c, r)
        wait_id(c, r)
        fire_s(c, r)

        @pl.when(c + 2 < _NB)
        def _():
          r2 = (r + 2) % _D

          @pl.when(c >= 2)
          def _():
            wait_s(c - 2, r2)

          fire_id(c + 2, r2)
          wait_is(c + 2, r2)
          fire_g(c + 2, r2)

        @pl.when(c + _D < _NB)
        def _():
          fire_is(c + _D, r)

  for r in range(_D):
    wait_s(0, r)

  plsc.subcore_barrier()

  @pl.loop(0, _NRB_PT)
  def _copy_out(k):
    blk = si + k * _NS

    @pl.when(blk < _NRB)
    def _():
      rr = pl.multiple_of(blk * _EB, _EB)
      pltpu.sync_copy(agg_sh.at[pl.ds(rr, _EB)],
                      agg_out.at[ci, pl.ds(rr, _EB)])


_sage_sc = pl.kernel(
    _sage_body,
    out_type=jax.ShapeDtypeStruct((_NC, N, W), jnp.float32),
    mesh=_mesh,
    scratch_types=[
        pltpu.VMEM((_D, _EB), jnp.int32),
        pltpu.VMEM((_D, _EB), jnp.int32),
        pltpu.VMEM((_D, _EB, W), jnp.float32),
        pltpu.VMEM_SHARED((N, W), jnp.float32),
    ] + [pltpu.SemaphoreType.DMA] * (4 * _D))


def _head_body(a_hbm, b_hbm, src3, dst3, w2_hbm, out_hbm, src_all, dst_all,
               ra, rb, tmp, lv, wv, *sems):
  sem_a = sems[:_DH]
  sem_b = sems[_DH:2 * _DH]
  sem_w = sems[2 * _DH:3 * _DH]
  ci = lax.axis_index("c")
  si = lax.axis_index("s")
  wid = ci * _NS + si
  pltpu.sync_copy(w2_hbm, wv)
  w = [wv[pl.ds(j * _L, _L)] for j in range(H // _L)]
  iota16 = lax.iota(jnp.int32, _L)
  pltpu.sync_copy(src3.at[wid], src_all)
  pltpu.sync_copy(dst3.at[wid], dst_all)
  base = wid * _EPT

  def fire_g(c, r):
    pltpu.async_copy(a_hbm.at[src_all.at[c]], ra.at[r], sem_a[r])
    pltpu.async_copy(b_hbm.at[dst_all.at[c]], rb.at[r], sem_b[r])

  def wait_g(c, r):
    pltpu.make_async_copy(a_hbm.at[src_all.at[c]], ra.at[r], sem_a[r]).wait()
    pltpu.make_async_copy(b_hbm.at[dst_all.at[c]], rb.at[r], sem_b[r]).wait()

  def wait_w(r):
    pltpu.make_async_copy(lv.at[r], out_hbm.at[pl.ds(0, _EB)],
                          sem_w[r]).wait()

  for r in range(_DH):
    fire_g(r, r)

  @pl.loop(0, -(-_NB // _DH))
  def _main(i):
    for r in range(_DH):
      c = i * _DH + r

      @pl.when(c < _NB)
      def _():
        wait_g(c, r)

        @pl.when(c >= _DH)
        def _():
          wait_w(r)

        @pl.loop(0, _EB // _L)
        def _group(g):
          for i2 in range(_L):
            acc0 = jnp.zeros((_L,), jnp.float32)
            acc1 = jnp.zeros((_L,), jnp.float32)
            for j in range(H // _L):
              cj = jnp.maximum(
                  ra[r, g * _L + i2, pl.ds(j * _L, _L)]
                  + rb[r, g * _L + i2, pl.ds(j * _L, _L)], 0.0)
              if j % 2 == 0:
                acc0 = acc0 + cj * w[j]
              else:
                acc1 = acc1 + cj * w[j]
            tmp[i2, pl.ds(0, _L)] = acc0 + acc1
          cols = [
              plsc.load_gather(tmp, [iota16, jnp.full((_L,), j, jnp.int32)])
              for j in range(_L)
          ]
          while len(cols) > 1:
            cols = [cols[k] + cols[k + 1] for k in range(0, len(cols), 2)]
          lv[r, pl.ds(g * _L, _L)] = cols[0]

        off = pl.multiple_of(base + c * _EB, _EB)
        pltpu.async_copy(lv.at[r], out_hbm.at[pl.ds(off, _EB)], sem_w[r])

        @pl.when(c + _DH < _NB)
        def _():
          fire_g(c + _DH, r)

  for r in range(_DH):
    wait_w(r)


_head_sc = pl.kernel(
    _head_body,
    out_type=jax.ShapeDtypeStruct((E,), jnp.float32),
    mesh=_mesh,
    compiler_params=_cp,
    scratch_types=[
        pltpu.VMEM((_NB, _EB), jnp.int32),
        pltpu.VMEM((_NB, _EB), jnp.int32),
        pltpu.VMEM((_DH, _EB, W), jnp.float32),
        pltpu.VMEM((_DH, _EB, W), jnp.float32),
        pltpu.VMEM((_L, _L), jnp.float32),
        pltpu.VMEM((_DH, _EB), jnp.float32),
        pltpu.VMEM((H,), jnp.float32),
    ] + [pltpu.SemaphoreType.DMA] * (3 * _DH))


def _pad_cols(h):
  n = h.shape[0]
  ind = (lax.broadcasted_iota(jnp.int32, (n, W - H), 1) == 0).astype(
      jnp.float32)
  return jnp.concatenate([h, ind], axis=1)


def _enc_body(x_ref, wn_ref, bn_ref, o_ref):
  h = jnp.maximum(
      jnp.dot(x_ref[...], wn_ref[...], preferred_element_type=jnp.float32)
      + bn_ref[...], 0.0)
  o_ref[...] = _pad_cols(h)


def _mean_agg(p0_ref, p1_ref):
  p = p0_ref[...] + p1_ref[...]
  deg = jnp.maximum(p[:, H:H + 1], 1.0)
  return p[:, :H] / deg


def _combine_body(p0, p1, hp, wl, bl, wr, o):
  mean = _mean_agg(p0, p1)
  h = jnp.maximum(
      jnp.dot(mean, wl[...], preferred_element_type=jnp.float32) + bl[...]
      + jnp.dot(hp[...][:, :H], wr[...], preferred_element_type=jnp.float32),
      0.0)
  o[...] = _pad_cols(h)


def _combine_head_body(p0, p1, hp, wl, bl, wr, wha, whb, bh, o_h, o_a, o_b):
  mean = _mean_agg(p0, p1)
  h2 = jnp.maximum(
      jnp.dot(mean, wl[...], preferred_element_type=jnp.float32) + bl[...]
      + jnp.dot(hp[...][:, :H], wr[...], preferred_element_type=jnp.float32),
      0.0)
  o_h[...] = h2
  a = jnp.dot(h2, wha[...], preferred_element_type=jnp.float32) + bh[...]
  b = jnp.dot(h2, whb[...], preferred_element_type=jnp.float32)
  zpad = jnp.zeros((h2.shape[0], W - H), jnp.float32)
  o_a[...] = jnp.concatenate([a, zpad], axis=1)
  o_b[...] = jnp.concatenate([b, zpad], axis=1)


def _final_body(l_ref, b_ref, o_s, o_p):
  s = jax.nn.sigmoid(l_ref[...] + b_ref[0, 0])
  o_s[...] = s
  o_p[...] = (jnp.sum(s) / E).reshape(1, 1)


def kernel(x, edge_index, edge_attr, Wn, bn, We, be, Wl0, bl0, Wr0, Wl1, bl1,
           Wr1, Wh1, bh1, Wh2, bh2):
  src = edge_index[0]
  dst = edge_index[1]
  f32 = jnp.float32

  h0p = pl.pallas_call(
      _enc_body, out_shape=jax.ShapeDtypeStruct((N, W), f32))(
          x, Wn, bn.reshape(1, H))

  agg0 = _sage_sc(h0p, src, dst)

  h1p = pl.pallas_call(
      _combine_body, out_shape=jax.ShapeDtypeStruct((N, W), f32))(
          agg0[0], agg0[1], h0p, Wl0, bl0.reshape(1, H), Wr0)

  agg1 = _sage_sc(h1p, src, dst)

  h2, ap, bp = pl.pallas_call(
      _combine_head_body,
      out_shape=[
          jax.ShapeDtypeStruct((N, H), f32),
          jax.ShapeDtypeStruct((N, W), f32),
          jax.ShapeDtypeStruct((N, W), f32),
      ])(agg1[0], agg1[1], h1p, Wl1, bl1.reshape(1, H), Wr1, Wh1[:H],
         Wh1[H:], bh1.reshape(1, H))

  logits = _head_sc(ap, bp, src.reshape(_NW, _NB, _EB),
                    dst.reshape(_NW, _NB, _EB), Wh2.reshape(H))

  scores2d, psum = pl.pallas_call(
      _final_body,
      out_shape=[
          jax.ShapeDtypeStruct((E // 128, 128), f32),
          jax.ShapeDtypeStruct((1, 1), f32),
      ])(logits.reshape(E // 128, 128), bh2.reshape(1, 1))

  return scores2d.reshape(E), psum[0, 0], h2


# speedup vs baseline: 1.0364x; 1.0284x over previous
import dataclasses

import jax
import jax.numpy as jnp
from jax import lax
from jax.experimental import pallas as pl
from jax.experimental.pallas import tpu as pltpu
from jax.experimental.pallas import tpu_sc as plsc

N = 10000
E = 320000
ND = 128
H = 64
W = 128

_NC = 2
_NS = 16
_NW = _NC * _NS
_EB = 80
_EPT = E // _NW
_NB = _EPT // _EB
_NRB = N // _EB
_NRB_PT = -(-_NRB // _NS)
_L = 16

_D = 4
_DH = 4

_mesh = plsc.VectorSubcoreMesh(core_axis_name="c", subcore_axis_name="s",
                               num_cores=_NC, num_subcores=_NS)

_cp = pltpu.CompilerParams()
if "needs_layout_passes" in pltpu.CompilerParams.__dataclass_fields__:
  _cp = dataclasses.replace(_cp, needs_layout_passes=False)


def _sage_body(h_hbm, src_hbm, dst_hbm, agg_out, src_r, dst_r, rows, agg_sh,
               *sems):
  sem_is = sems[:_D]
  sem_id = sems[_D:2 * _D]
  sem_g = sems[2 * _D:3 * _D]
  sem_s = sems[3 * _D:4 * _D]
  ci = lax.axis_index("c")
  si = lax.axis_index("s")
  wid = ci * _NS + si

  @pl.loop(0, _EB)
  def _zero_rows(r):
    for j in range(W // _L):
      rows[0, r, pl.ds(j * _L, _L)] = jnp.zeros((_L,), jnp.float32)

  @pl.loop(0, _NRB_PT)
  def _zero_sh(k):
    blk = si + k * _NS

    @pl.when(blk < _NRB)
    def _():
      rr = pl.multiple_of(blk * _EB, _EB)
      pltpu.sync_copy(rows.at[0], agg_sh.at[pl.ds(rr, _EB)])

  def _off(c):
    return pl.multiple_of(wid * _EPT + c * _EB, _EB)

  def fire_is(c, r):
    pltpu.async_copy(src_hbm.at[pl.ds(_off(c), _EB)], src_r.at[r], sem_is[r])

  def wait_is(c, r):
    pltpu.make_async_copy(src_hbm.at[pl.ds(_off(c), _EB)], src_r.at[r],
                          sem_is[r]).wait()

  def fire_id(c, r):
    pltpu.async_copy(dst_hbm.at[pl.ds(_off(c), _EB)], dst_r.at[r], sem_id[r])

  def wait_id(c, r):
    pltpu.make_async_copy(dst_hbm.at[pl.ds(_off(c), _EB)], dst_r.at[r],
                          sem_id[r]).wait()

  def fire_g(c, r):
    pltpu.async_copy(h_hbm.at[src_r.at[r]], rows.at[r], sem_g[r])

  def wait_g(c, r):
    pltpu.make_async_copy(h_hbm.at[src_r.at[r]], rows.at[r], sem_g[r]).wait()

  def fire_s(c, r):
    pltpu.async_copy(rows.at[r], agg_sh.at[dst_r.at[r]], sem_s[r], add=True)

  def wait_s(c, r):
    pltpu.make_async_copy(rows.at[r], agg_sh.at[dst_r.at[r]],
                          sem_s[r]).wait()

  for r in range(_D):
    fire_is(r, r)
  for r in range(2):
    fire_id(r, r)
    wait_is(r, r)
    fire_g(r, r)

  plsc.subcore_barrier()

  @pl.loop(0, -(-_NB // _D))
  def _main(i):
    for r in range(_D):
      c = i * _D + r

      @pl.when(c < _NB)
      def _():
        wait_g(c, r)
        wait_id(c, r)
        fire_s(c, r)

        @pl.when(c + 2 < _NB)
        def _():
          r2 = (r + 2) % _D

          @pl.when(c >= 2)
          def _():
            wait_s(c - 2, r2)

          fire_id(c + 2, r2)
          wait_is(c + 2, r2)
          fire_g(c + 2, r2)

        @pl.when(c + _D < _NB)
        def _():
          fire_is(c + _D, r)

  for r in range(_D):
    wait_s(0, r)

  plsc.subcore_barrier()

  @pl.loop(0, _NRB_PT)
  def _copy_out(k):
    blk = si + k * _NS

    @pl.when(blk < _NRB)
    def _():
      rr = pl.multiple_of(blk * _EB, _EB)
      pltpu.sync_copy(agg_sh.at[pl.ds(rr, _EB)],
                      agg_out.at[ci, pl.ds(rr, _EB)])


_sage_sc = pl.kernel(
    _sage_body,
    out_type=jax.ShapeDtypeStruct((_NC, N, W), jnp.float32),
    mesh=_mesh,
    scratch_types=[
        pltpu.VMEM((_D, _EB), jnp.int32),
        pltpu.VMEM((_D, _EB), jnp.int32),
        pltpu.VMEM((_D, _EB, W), jnp.float32),
        pltpu.VMEM_SHARED((N, W), jnp.float32),
    ] + [pltpu.SemaphoreType.DMA] * (4 * _D))


def _head_body(a_hbm, b_hbm, src3, dst3, w2_hbm, out_hbm, src_all, dst_all,
               ra, rb, tmp, lv, wv, *sems):
  sem_a = sems[:_DH]
  sem_b = sems[_DH:2 * _DH]
  sem_w = sems[2 * _DH:3 * _DH]
  ci = lax.axis_index("c")
  si = lax.axis_index("s")
  wid = ci * _NS + si
  pltpu.sync_copy(w2_hbm, wv)
  w = [wv[pl.ds(j * _L, _L)] for j in range(H // _L)]
  iota16 = lax.iota(jnp.int32, _L)
  pltpu.sync_copy(src3.at[wid], src_all)
  pltpu.sync_copy(dst3.at[wid], dst_all)
  base = wid * _EPT

  def fire_g(c, r):
    pltpu.async_copy(a_hbm.at[src_all.at[c]], ra.at[r], sem_a[r])
    pltpu.async_copy(b_hbm.at[dst_all.at[c]], rb.at[r], sem_b[r])

  def wait_g(c, r):
    pltpu.make_async_copy(a_hbm.at[src_all.at[c]], ra.at[r], sem_a[r]).wait()
    pltpu.make_async_copy(b_hbm.at[dst_all.at[c]], rb.at[r], sem_b[r]).wait()

  def wait_w(r):
    pltpu.make_async_copy(lv.at[r], out_hbm.at[pl.ds(0, _EB)],
                          sem_w[r]).wait()

  for r in range(_DH):
    fire_g(r, r)

  @pl.loop(0, -(-_NB // _DH))
  def _main(i):
    for r in range(_DH):
      c = i * _DH + r

      @pl.when(c < _NB)
      def _():
        wait_g(c, r)

        @pl.when(c >= _DH)
        def _():
          wait_w(r)

        @pl.loop(0, _EB // _L)
        def _group(g):
          for i2 in range(_L):
            acc = jnp.zeros((_L,), jnp.float32)
            for j in range(H // _L):
              cj = jnp.maximum(
                  ra[r, g * _L + i2, pl.ds(j * _L, _L)]
                  + rb[r, g * _L + i2, pl.ds(j * _L, _L)], 0.0)
              acc = acc + cj * w[j]
            tmp[i2, pl.ds(0, _L)] = acc
          tot = jnp.zeros((_L,), jnp.float32)
          for j in range(_L):
            tot = tot + plsc.load_gather(
                tmp, [iota16, jnp.full((_L,), j, jnp.int32)])
          lv[r, pl.ds(g * _L, _L)] = tot

        off = pl.multiple_of(base + c * _EB, _EB)
        pltpu.async_copy(lv.at[r], out_hbm.at[pl.ds(off, _EB)], sem_w[r])

        @pl.when(c + _DH < _NB)
        def _():
          fire_g(c + _DH, r)

  for r in range(_DH):
    wait_w(r)


_head_sc = pl.kernel(
    _head_body,
    out_type=jax.ShapeDtypeStruct((E,), jnp.float32),
    mesh=_mesh,
    compiler_params=_cp,
    scratch_types=[
        pltpu.VMEM((_NB, _EB), jnp.int32),
        pltpu.VMEM((_NB, _EB), jnp.int32),
        pltpu.VMEM((_DH, _EB, W), jnp.float32),
        pltpu.VMEM((_DH, _EB, W), jnp.float32),
        pltpu.VMEM((_L, _L), jnp.float32),
        pltpu.VMEM((_DH, _EB), jnp.float32),
        pltpu.VMEM((H,), jnp.float32),
    ] + [pltpu.SemaphoreType.DMA] * (3 * _DH))


def _pad_cols(h):
  n = h.shape[0]
  ind = (lax.broadcasted_iota(jnp.int32, (n, W - H), 1) == 0).astype(
      jnp.float32)
  return jnp.concatenate([h, ind], axis=1)


def _enc_body(x_ref, wn_ref, bn_ref, o_ref):
  h = jnp.maximum(
      jnp.dot(x_ref[...], wn_ref[...], preferred_element_type=jnp.float32)
      + bn_ref[...], 0.0)
  o_ref[...] = _pad_cols(h)


def _mean_agg(p0_ref, p1_ref):
  p = p0_ref[...] + p1_ref[...]
  deg = jnp.maximum(p[:, H:H + 1], 1.0)
  return p[:, :H] / deg


def _combine_body(p0, p1, hp, wl, bl, wr, o):
  mean = _mean_agg(p0, p1)
  h = jnp.maximum(
      jnp.dot(mean, wl[...], preferred_element_type=jnp.float32) + bl[...]
      + jnp.dot(hp[...][:, :H], wr[...], preferred_element_type=jnp.float32),
      0.0)
  o[...] = _pad_cols(h)


def _combine_head_body(p0, p1, hp, wl, bl, wr, wha, whb, bh, o_h, o_a, o_b):
  mean = _mean_agg(p0, p1)
  h2 = jnp.maximum(
      jnp.dot(mean, wl[...], preferred_element_type=jnp.float32) + bl[...]
      + jnp.dot(hp[...][:, :H], wr[...], preferred_element_type=jnp.float32),
      0.0)
  o_h[...] = h2
  a = jnp.dot(h2, wha[...], preferred_element_type=jnp.float32) + bh[...]
  b = jnp.dot(h2, whb[...], preferred_element_type=jnp.float32)
  zpad = jnp.zeros((h2.shape[0], W - H), jnp.float32)
  o_a[...] = jnp.concatenate([a, zpad], axis=1)
  o_b[...] = jnp.concatenate([b, zpad], axis=1)


def _final_body(l_ref, b_ref, o_s, o_p):
  s = jax.nn.sigmoid(l_ref[...] + b_ref[0, 0])
  o_s[...] = s
  o_p[...] = (jnp.sum(s) / E).reshape(1, 1)


def kernel(x, edge_index, edge_attr, Wn, bn, We, be, Wl0, bl0, Wr0, Wl1, bl1,
           Wr1, Wh1, bh1, Wh2, bh2):
  src = edge_index[0]
  dst = edge_index[1]
  f32 = jnp.float32

  h0p = pl.pallas_call(
      _enc_body, out_shape=jax.ShapeDtypeStruct((N, W), f32))(
          x, Wn, bn.reshape(1, H))

  agg0 = _sage_sc(h0p, src, dst)

  h1p = pl.pallas_call(
      _combine_body, out_shape=jax.ShapeDtypeStruct((N, W), f32))(
          agg0[0], agg0[1], h0p, Wl0, bl0.reshape(1, H), Wr0)

  agg1 = _sage_sc(h1p, src, dst)

  h2, ap, bp = pl.pallas_call(
      _combine_head_body,
      out_shape=[
          jax.ShapeDtypeStruct((N, H), f32),
          jax.ShapeDtypeStruct((N, W), f32),
          jax.ShapeDtypeStruct((N, W), f32),
      ])(agg1[0], agg1[1], h1p, Wl1, bl1.reshape(1, H), Wr1, Wh1[:H],
         Wh1[H:], bh1.reshape(1, H))

  logits = _head_sc(ap, bp, src.reshape(_NW, _NB, _EB),
                    dst.reshape(_NW, _NB, _EB), Wh2.reshape(H))

  scores2d, psum = pl.pallas_call(
      _final_body,
      out_shape=[
          jax.ShapeDtypeStruct((E // 128, 128), f32),
          jax.ShapeDtypeStruct((1, 1), f32),
      ])(logits.reshape(E // 128, 128), bh2.reshape(1, 1))

  return scores2d.reshape(E), psum[0, 0], h2
